# baseline (device time: 415868 ns/iter reference)
import jax
import jax.numpy as jnp
from jax import lax
from jax.experimental import pallas as pl
from jax.experimental.pallas import tpu as pltpu

N_DEV = 4
S_LOC = 2048
D = 1024
HL = 8
DH = 128
SKV = 2048
Q_TILE = 256
N_TILES = S_LOC // Q_TILE
SCALE = 0.08838834764831843
NEG = -1e9

_CompilerParams = getattr(pltpu, "CompilerParams", None) or pltpu.TPUCompilerParams


def kernel(x, Wq, K_ext, V_ext, Wo):
    j = lax.axis_index("i")
    xb = x[0].astype(jnp.bfloat16)
    wq = (Wq * SCALE).astype(jnp.bfloat16)
    wo = Wo.astype(jnp.bfloat16)
    k_loc = lax.dynamic_slice_in_dim(K_ext[0], j * HL, HL, axis=1)
    v_loc = lax.dynamic_slice_in_dim(V_ext[0], j * HL, HL, axis=1)
    kt = jnp.transpose(k_loc, (1, 2, 0)).astype(jnp.bfloat16)
    vt = jnp.transpose(v_loc, (1, 0, 2)).astype(jnp.bfloat16)

    def body(x_ref, wq_ref, kt_ref, vt_ref, wo_ref, out_ref,
             xg_ref, rs_ref, pmy_ref, tmp_ref,
             ag_send, ag_recv, rs_send, rs_recv):
        my = lax.axis_index("i")
        right = lax.rem(my + 1, N_DEV)
        left = lax.rem(my + N_DEV - 1, N_DEV)

        barrier_sem = pltpu.get_barrier_semaphore()
        for nbr in (left, right):
            pl.semaphore_signal(barrier_sem, inc=1, device_id=(nbr,),
                                device_id_type=pl.DeviceIdType.MESH)
        pl.semaphore_wait(barrier_sem, 2)

        def p_tile(loader, row0, c):
            x_t = loader(row0)
            q = jnp.dot(x_t, wq_ref[...],
                        preferred_element_type=jnp.float32).astype(jnp.bfloat16)
            qi = lax.broadcasted_iota(jnp.int32, (Q_TILE, SKV), 0)
            ki = lax.broadcasted_iota(jnp.int32, (Q_TILE, SKV), 1)
            qb = (c * S_LOC + row0 + qi) // 64
            kb = ki // 64
            mask = (qb == kb) | (kb == 0) | ((qb + kb) % 3 == 0)
            bias = jnp.where(mask, 0.0, NEG).astype(jnp.bfloat16)
            parts = []
            for h in range(HL):
                qh = q[:, h * DH:(h + 1) * DH]
                s = jnp.dot(qh, kt_ref[h],
                            preferred_element_type=jnp.float32
                            ).astype(jnp.bfloat16) + bias
                w = jnp.exp(s)
                d = jnp.sum(w, axis=1, keepdims=True, dtype=jnp.float32)
                w = w * (1.0 / d).astype(jnp.bfloat16)
                parts.append(jnp.dot(w, vt_ref[h],
                                     preferred_element_type=jnp.float32))
            ctx = jnp.concatenate(parts, axis=1).astype(jnp.bfloat16)
            return jnp.dot(ctx, wo_ref[...], preferred_element_type=jnp.float32)

        def chunk_into(loader, c, dst_ref):
            def tbody(t, _):
                row0 = t * Q_TILE
                dst_ref[pl.ds(row0, Q_TILE), :] = \
                    p_tile(loader, row0, c).astype(jnp.bfloat16)
                return 0
            lax.fori_loop(0, N_TILES, tbody, 0)

        def ag_rdma(h):
            src = x_ref if h == 0 else xg_ref.at[h - 1]
            return pltpu.make_async_remote_copy(
                src_ref=src, dst_ref=xg_ref.at[h],
                send_sem=ag_send.at[h], recv_sem=ag_recv.at[h],
                device_id=(right,), device_id_type=pl.DeviceIdType.MESH)

        def rs_rdma(s):
            dst = out_ref if s == N_DEV - 2 else rs_ref.at[s + 1]
            return pltpu.make_async_remote_copy(
                src_ref=rs_ref.at[s], dst_ref=dst,
                send_sem=rs_send.at[s], recv_sem=rs_recv.at[s],
                device_id=(right,), device_id_type=pl.DeviceIdType.MESH)

        ag = [ag_rdma(h) for h in range(N_DEV - 1)]
        rs = [rs_rdma(s) for s in range(N_DEV - 1)]

        c = [lax.rem(my - 1 - s + 2 * N_DEV, N_DEV) for s in range(N_DEV - 1)]

        ag[0].start()
        chunk_into(lambda r0: x_ref[pl.ds(r0, Q_TILE), :], my, pmy_ref)

        ag[0].wait_recv()
        ag[1].start()
        chunk_into(lambda r0: xg_ref[0, pl.ds(r0, Q_TILE), :], c[0], rs_ref.at[0])
        rs[0].start()

        ag[1].wait_recv()
        ag[2].start()
        chunk_into(lambda r0: xg_ref[1, pl.ds(r0, Q_TILE), :], c[1], tmp_ref)
        rs[0].wait_recv()
        rs_ref[1] = (rs_ref[1].astype(jnp.float32)
                     + tmp_ref[...].astype(jnp.float32)).astype(jnp.bfloat16)
        rs[1].start()

        ag[2].wait_recv()
        chunk_into(lambda r0: xg_ref[2, pl.ds(r0, Q_TILE), :], c[2], tmp_ref)
        rs[1].wait_recv()
        rs_ref[2] = (rs_ref[2].astype(jnp.float32)
                     + tmp_ref[...].astype(jnp.float32)).astype(jnp.bfloat16)
        rs[2].start()

        rs[2].wait_recv()
        out_ref[...] = (out_ref[...].astype(jnp.float32)
                        + pmy_ref[...].astype(jnp.float32)).astype(jnp.bfloat16)

        for r in ag + rs:
            r.wait_send()

    out = pl.pallas_call(
        body,
        out_shape=jax.ShapeDtypeStruct((S_LOC, D), jnp.bfloat16),
        in_specs=[pl.BlockSpec(memory_space=pltpu.VMEM)] * 5,
        out_specs=pl.BlockSpec(memory_space=pltpu.VMEM),
        scratch_shapes=[
            pltpu.VMEM((N_DEV - 1, S_LOC, D), jnp.bfloat16),
            pltpu.VMEM((N_DEV - 1, S_LOC, D), jnp.bfloat16),
            pltpu.VMEM((S_LOC, D), jnp.bfloat16),
            pltpu.VMEM((S_LOC, D), jnp.bfloat16),
            pltpu.SemaphoreType.DMA((N_DEV - 1,)),
            pltpu.SemaphoreType.DMA((N_DEV - 1,)),
            pltpu.SemaphoreType.DMA((N_DEV - 1,)),
            pltpu.SemaphoreType.DMA((N_DEV - 1,)),
        ],
        compiler_params=_CompilerParams(collective_id=0,
                                        vmem_limit_bytes=60 * 1024 * 1024),
    )(xb, wq, kt, vt, wo)
    return out.astype(jnp.float32)[None]


# device time: 371018 ns/iter; 1.1209x vs baseline; 1.1209x over previous
import jax
import jax.numpy as jnp
from jax import lax
from jax.experimental import pallas as pl
from jax.experimental.pallas import tpu as pltpu

N_DEV = 4
S_LOC = 2048
D = 1024
HL = 8
DH = 128
SKV = 2048
Q_TILE = 256
N_TILES = S_LOC // Q_TILE
SCALE = 0.08838834764831843
NEG = -1e9

_CompilerParams = getattr(pltpu, "CompilerParams", None) or pltpu.TPUCompilerParams


def kernel(x, Wq, K_ext, V_ext, Wo):
    j = lax.axis_index("i")
    xb = x[0].astype(jnp.bfloat16)
    wq = (Wq * SCALE).astype(jnp.bfloat16)
    wo = Wo.astype(jnp.bfloat16)
    k_loc = lax.dynamic_slice_in_dim(K_ext[0], j * HL, HL, axis=1)
    v_loc = lax.dynamic_slice_in_dim(V_ext[0], j * HL, HL, axis=1)
    kt = jnp.transpose(k_loc, (1, 2, 0)).astype(jnp.bfloat16)
    vt = jnp.transpose(v_loc, (1, 0, 2)).astype(jnp.bfloat16)
    vt = jnp.concatenate(
        [vt, jnp.ones((HL, SKV, 1), jnp.bfloat16),
         jnp.zeros((HL, SKV, 127), jnp.bfloat16)], axis=2)

    def body(x_ref, wq_ref, kt_ref, vt_ref, wo_ref, out_ref,
             xg_ref, rs_ref, pmy_ref, tmp_ref,
             ag_send, ag_recv, rs_send, rs_recv):
        my = lax.axis_index("i")
        right = lax.rem(my + 1, N_DEV)
        left = lax.rem(my + N_DEV - 1, N_DEV)

        barrier_sem = pltpu.get_barrier_semaphore()
        for nbr in (left, right):
            pl.semaphore_signal(barrier_sem, inc=1, device_id=(nbr,),
                                device_id_type=pl.DeviceIdType.MESH)
        pl.semaphore_wait(barrier_sem, 2)

        def p_tile(loader, row0, c):
            x_t = loader(row0)
            q = jnp.dot(x_t, wq_ref[...],
                        preferred_element_type=jnp.float32).astype(jnp.bfloat16)
            qi = lax.broadcasted_iota(jnp.int32, (Q_TILE, SKV), 0)
            ki = lax.broadcasted_iota(jnp.int32, (Q_TILE, SKV), 1)
            qb = (c * S_LOC + row0 + qi) // 64
            kb = ki // 64
            mask = (qb == kb) | (kb == 0) | ((qb + kb) % 3 == 0)
            bias = jnp.where(mask, 0.0, NEG)
            parts = []
            for h in range(HL):
                qh = q[:, h * DH:(h + 1) * DH]
                s = jnp.dot(qh, kt_ref[h],
                            preferred_element_type=jnp.float32) + bias
                w = jnp.exp(s).astype(jnp.bfloat16)
                pa = jnp.dot(w, vt_ref[h],
                             preferred_element_type=jnp.float32)
                parts.append(pa[:, :DH] * (1.0 / pa[:, DH:DH + 1]))
            ctx = jnp.concatenate(parts, axis=1).astype(jnp.bfloat16)
            return jnp.dot(ctx, wo_ref[...], preferred_element_type=jnp.float32)

        def chunk_into(loader, c, dst_ref):
            def tbody(t, _):
                row0 = t * Q_TILE
                dst_ref[pl.ds(row0, Q_TILE), :] = \
                    p_tile(loader, row0, c).astype(jnp.bfloat16)
                return 0
            lax.fori_loop(0, N_TILES, tbody, 0)

        def ag_rdma(h):
            src = x_ref if h == 0 else xg_ref.at[h - 1]
            return pltpu.make_async_remote_copy(
                src_ref=src, dst_ref=xg_ref.at[h],
                send_sem=ag_send.at[h], recv_sem=ag_recv.at[h],
                device_id=(right,), device_id_type=pl.DeviceIdType.MESH)

        def rs_rdma(s):
            dst = out_ref if s == N_DEV - 2 else rs_ref.at[s + 1]
            return pltpu.make_async_remote_copy(
                src_ref=rs_ref.at[s], dst_ref=dst,
                send_sem=rs_send.at[s], recv_sem=rs_recv.at[s],
                device_id=(right,), device_id_type=pl.DeviceIdType.MESH)

        ag = [ag_rdma(h) for h in range(N_DEV - 1)]
        rs = [rs_rdma(s) for s in range(N_DEV - 1)]

        c = [lax.rem(my - 1 - s + 2 * N_DEV, N_DEV) for s in range(N_DEV - 1)]

        ag[0].start()
        chunk_into(lambda r0: x_ref[pl.ds(r0, Q_TILE), :], my, pmy_ref)

        ag[0].wait_recv()
        ag[1].start()
        chunk_into(lambda r0: xg_ref[0, pl.ds(r0, Q_TILE), :], c[0], rs_ref.at[0])
        rs[0].start()

        ag[1].wait_recv()
        ag[2].start()
        chunk_into(lambda r0: xg_ref[1, pl.ds(r0, Q_TILE), :], c[1], tmp_ref)
        rs[0].wait_recv()
        rs_ref[1] = (rs_ref[1].astype(jnp.float32)
                     + tmp_ref[...].astype(jnp.float32)).astype(jnp.bfloat16)
        rs[1].start()

        ag[2].wait_recv()
        chunk_into(lambda r0: xg_ref[2, pl.ds(r0, Q_TILE), :], c[2], tmp_ref)
        rs[1].wait_recv()
        rs_ref[2] = (rs_ref[2].astype(jnp.float32)
                     + tmp_ref[...].astype(jnp.float32)).astype(jnp.bfloat16)
        rs[2].start()

        rs[2].wait_recv()
        out_ref[...] = (out_ref[...].astype(jnp.float32)
                        + pmy_ref[...].astype(jnp.float32)).astype(jnp.bfloat16)

        for r in ag + rs:
            r.wait_send()

    out = pl.pallas_call(
        body,
        out_shape=jax.ShapeDtypeStruct((S_LOC, D), jnp.bfloat16),
        in_specs=[pl.BlockSpec(memory_space=pltpu.VMEM)] * 5,
        out_specs=pl.BlockSpec(memory_space=pltpu.VMEM),
        scratch_shapes=[
            pltpu.VMEM((N_DEV - 1, S_LOC, D), jnp.bfloat16),
            pltpu.VMEM((N_DEV - 1, S_LOC, D), jnp.bfloat16),
            pltpu.VMEM((S_LOC, D), jnp.bfloat16),
            pltpu.VMEM((S_LOC, D), jnp.bfloat16),
            pltpu.SemaphoreType.DMA((N_DEV - 1,)),
            pltpu.SemaphoreType.DMA((N_DEV - 1,)),
            pltpu.SemaphoreType.DMA((N_DEV - 1,)),
            pltpu.SemaphoreType.DMA((N_DEV - 1,)),
        ],
        compiler_params=_CompilerParams(collective_id=0,
                                        vmem_limit_bytes=62 * 1024 * 1024),
    )(xb, wq, kt, vt, wo)
    return out.astype(jnp.float32)[None]


# device time: 369107 ns/iter; 1.1267x vs baseline; 1.0052x over previous
import jax
import jax.numpy as jnp
from jax import lax
from jax.experimental import pallas as pl
from jax.experimental.pallas import tpu as pltpu

N_DEV = 4
S_LOC = 2048
D = 1024
HL = 8
DH = 128
SKV = 2048
NB = 32
GB = 12
GR = GB * 64
GW = 896
SCALE = 0.08838834764831843
NEG = -1e9

_CompilerParams = getattr(pltpu, "CompilerParams", None) or pltpu.TPUCompilerParams


def kernel(x, Wq, K_ext, V_ext, Wo):
    j = lax.axis_index("i")
    xb = x[0].astype(jnp.bfloat16)
    wq = (Wq * SCALE).astype(jnp.bfloat16)
    wo = Wo.astype(jnp.bfloat16)
    k_loc = lax.dynamic_slice_in_dim(K_ext[0], j * HL, HL, axis=1)
    v_loc = lax.dynamic_slice_in_dim(V_ext[0], j * HL, HL, axis=1)
    kth = jnp.transpose(k_loc, (1, 2, 0)).astype(jnp.bfloat16)
    vth = jnp.transpose(v_loc, (1, 0, 2)).astype(jnp.bfloat16)
    kb_ = kth.reshape(HL, DH, NB, 64)
    vb_ = vth.reshape(HL, NB, 64, DH)

    def v_aug(vrows):
        n = vrows.shape[1]
        return jnp.concatenate(
            [vrows, jnp.ones((HL, n, 1), jnp.bfloat16),
             jnp.zeros((HL, n, 127), jnp.bfloat16)], axis=2)

    kz = jnp.zeros((HL, DH, 64), jnp.bfloat16)
    vz = jnp.zeros((HL, 64, 256), jnp.bfloat16)
    ktg_l, vg_l = [], []
    for rho in range(3):
        bs = [b for b in range(NB) if b % 3 == rho]
        kcols = [kb_[:, :, b, :] for b in bs] + [kz] * (GB - len(bs))
        vrows = [v_aug(vb_[:, b]) for b in bs] + [vz] * (GB - len(bs))
        kcols += [kz if rho == 0 else kb_[:, :, 0, :], kz]
        vrows += [vz if rho == 0 else v_aug(vb_[:, 0]), vz]
        ktg_l.append(jnp.concatenate(kcols, axis=2))
        vg_l.append(jnp.concatenate(vrows, axis=1))
    ktg = jnp.stack(ktg_l, axis=1)
    vg = jnp.stack(vg_l, axis=1)

    def body(x_ref, wq_ref, ktg_ref, vg_ref, wo_ref, out_ref, xg_ref,
             rs_ref, pmy_ref, tmp_ref, xq_ref, pg_ref, pa_ref,
             ag_send, ag_recv, rs_send, rs_recv, gsem):
        my = lax.axis_index("i")
        right = lax.rem(my + 1, N_DEV)
        left = lax.rem(my + N_DEV - 1, N_DEV)

        barrier_sem = pltpu.get_barrier_semaphore()
        for nbr in (left, right):
            pl.semaphore_signal(barrier_sem, inc=1, device_id=(nbr,),
                                device_id_type=pl.DeviceIdType.MESH)
        pl.semaphore_wait(barrier_sem, 2)

        def sparse_group(c, g, mk_src, store_block):
            rho = (3 - g) % 3
            b0 = lax.rem(g - 2 * c + 12, 3)
            cps = []
            for i in range(GB):
                b = jnp.minimum(b0 + 3 * i, NB - 1)
                cp = pltpu.make_async_copy(
                    mk_src(b), xq_ref.at[pl.ds(i * 64, 64), :], gsem.at[i])
                cp.start()
                cps.append(cp)
            for cp in cps:
                cp.wait()
            qg = jnp.dot(xq_ref[...], wq_ref[...],
                         preferred_element_type=jnp.float32
                         ).astype(jnp.bfloat16)
            parts = []
            for h in range(HL):
                qh = qg[:, h * DH:(h + 1) * DH]
                s = jnp.dot(qh, ktg_ref[h, rho],
                            preferred_element_type=jnp.float32)
                w = jnp.exp(s).astype(jnp.bfloat16)
                pa_ref[...] = jnp.dot(w, vg_ref[h, rho],
                                      preferred_element_type=jnp.float32)
                if g != 0:
                    @pl.when(c == 0)
                    def _():
                        ri = lax.broadcasted_iota(jnp.int32, (GR, GW), 0)
                        cj = lax.broadcasted_iota(jnp.int32, (GR, GW), 1)
                        dbias = jnp.where(ri // 64 == cj // 64, 0.0, NEG)
                        sd = jnp.dot(qh, ktg_ref[h, g],
                                     preferred_element_type=jnp.float32)
                        wd = jnp.exp(sd + dbias).astype(jnp.bfloat16)
                        pa_ref[...] = pa_ref[...] + jnp.dot(
                            wd, vg_ref[h, g],
                            preferred_element_type=jnp.float32)
                pa = pa_ref[...]
                parts.append(pa[:, :DH] * (1.0 / pa[:, DH:DH + 1]))
            ctx = jnp.concatenate(parts, axis=1).astype(jnp.bfloat16)
            pg_ref[...] = jnp.dot(ctx, wo_ref[...],
                                  preferred_element_type=jnp.float32
                                  ).astype(jnp.bfloat16)

            def sbody(i, _):
                b = b0 + 3 * i
                @pl.when(b < NB)
                def _():
                    store_block(b, i)
                return 0
            lax.fori_loop(0, GB, sbody, 0)

        def compute_chunk(c, mk_src, dst_ref, lead=None):
            if lead is None:
                def store_block(b, i):
                    dst_ref[pl.ds(b * 64, 64), :] = pg_ref[pl.ds(i * 64, 64), :]
            else:
                def store_block(b, i):
                    dst_ref[lead, pl.ds(b * 64, 64), :] = \
                        pg_ref[pl.ds(i * 64, 64), :]
            for g in range(3):
                sparse_group(c, g, mk_src, store_block)

        def ag_rdma(h):
            src = x_ref if h == 0 else xg_ref.at[h - 1]
            return pltpu.make_async_remote_copy(
                src_ref=src, dst_ref=xg_ref.at[h],
                send_sem=ag_send.at[h], recv_sem=ag_recv.at[h],
                device_id=(right,), device_id_type=pl.DeviceIdType.MESH)

        def rs_rdma(s):
            dst = out_ref if s == N_DEV - 2 else rs_ref.at[s + 1]
            return pltpu.make_async_remote_copy(
                src_ref=rs_ref.at[s], dst_ref=dst,
                send_sem=rs_send.at[s], recv_sem=rs_recv.at[s],
                device_id=(right,), device_id_type=pl.DeviceIdType.MESH)

        ag = [ag_rdma(h) for h in range(N_DEV - 1)]
        rs = [rs_rdma(s) for s in range(N_DEV - 1)]

        c = [lax.rem(my - 1 - s + 2 * N_DEV, N_DEV) for s in range(N_DEV - 1)]

        def xg_src(slot):
            return lambda b: xg_ref.at[slot, pl.ds(b * 64, 64), :]

        ag[0].start()
        compute_chunk(my, lambda b: x_ref.at[pl.ds(b * 64, 64), :], pmy_ref)

        ag[0].wait_recv()
        ag[1].start()
        compute_chunk(c[0], xg_src(0), rs_ref, lead=0)
        rs[0].start()

        ag[1].wait_recv()
        ag[2].start()
        compute_chunk(c[1], xg_src(1), tmp_ref)
        rs[0].wait_recv()
        rs_ref[1] = (rs_ref[1].astype(jnp.float32)
                     + tmp_ref[...].astype(jnp.float32)).astype(jnp.bfloat16)
        rs[1].start()

        ag[2].wait_recv()
        compute_chunk(c[2], xg_src(2), tmp_ref)
        rs[1].wait_recv()
        rs_ref[2] = (rs_ref[2].astype(jnp.float32)
                     + tmp_ref[...].astype(jnp.float32)).astype(jnp.bfloat16)
        rs[2].start()

        rs[2].wait_recv()
        out_ref[...] = (out_ref[...].astype(jnp.float32)
                        + pmy_ref[...].astype(jnp.float32)).astype(jnp.bfloat16)

        for r in ag + rs:
            r.wait_send()

    out, _ = pl.pallas_call(
        body,
        out_shape=(
            jax.ShapeDtypeStruct((S_LOC, D), jnp.bfloat16),
            jax.ShapeDtypeStruct((N_DEV - 1, S_LOC, D), jnp.bfloat16),
        ),
        in_specs=[pl.BlockSpec(memory_space=pltpu.VMEM)] * 5,
        out_specs=(pl.BlockSpec(memory_space=pltpu.VMEM),
                   pl.BlockSpec(memory_space=pltpu.HBM)),
        scratch_shapes=[
            pltpu.VMEM((N_DEV - 1, S_LOC, D), jnp.bfloat16),
            pltpu.VMEM((S_LOC, D), jnp.bfloat16),
            pltpu.VMEM((S_LOC, D), jnp.bfloat16),
            pltpu.VMEM((GR, D), jnp.bfloat16),
            pltpu.VMEM((GR, D), jnp.bfloat16),
            pltpu.VMEM((GR, 256), jnp.float32),
            pltpu.SemaphoreType.DMA((N_DEV - 1,)),
            pltpu.SemaphoreType.DMA((N_DEV - 1,)),
            pltpu.SemaphoreType.DMA((N_DEV - 1,)),
            pltpu.SemaphoreType.DMA((N_DEV - 1,)),
            pltpu.SemaphoreType.DMA((GB,)),
        ],
        compiler_params=_CompilerParams(collective_id=0,
                                        vmem_limit_bytes=62 * 1024 * 1024),
    )(xb, wq, ktg, vg, wo)
    return out.astype(jnp.float32)[None]
